# baseline (speedup 1.0000x reference)
import functools

import jax
import jax.numpy as jnp
from jax import lax
from jax.experimental import pallas as pl
from jax.experimental.pallas import tpu as pltpu
from jax.experimental.pallas import tpu_sc as plsc

CHUNK = 128
N_TILES = 16
LANES = 16
RING = 4


def _dense_in(x, W_all, b_all):
    n, d = x.shape
    ko = W_all.shape[1]
    bn = 1000

    def body(x_ref, w_ref, b_ref, outa_ref, outb_ref):
        h = jnp.dot(x_ref[...], w_ref[...],
                    preferred_element_type=jnp.float32) + b_ref[...]
        outa_ref[...] = h[:, : ko // 2]
        outb_ref[...] = h[:, ko // 2:]

    return pl.pallas_call(
        body,
        grid=(n // bn,),
        in_specs=[
            pl.BlockSpec((bn, d), lambda i: (i, 0)),
            pl.BlockSpec((d, ko), lambda i: (0, 0)),
            pl.BlockSpec((1, ko), lambda i: (0, 0)),
        ],
        out_specs=[
            pl.BlockSpec((bn, ko // 2), lambda i: (i, 0)),
            pl.BlockSpec((bn, ko // 2), lambda i: (i, 0)),
        ],
        out_shape=[
            jax.ShapeDtypeStruct((n, ko // 2), jnp.float32),
            jax.ShapeDtypeStruct((n, ko // 2), jnp.float32),
        ],
    )(x, W_all, b_all)


def _dense_mid(g1, g2, g3, s1, s2, s3):
    n = g1.shape[0]
    bn = n // 16

    def body(g1_ref, g2_ref, g3_ref, s1_ref, s2_ref, s3_ref,
             outa_ref, outb_ref):
        outa_ref[...] = jnp.concatenate(
            [
                jnp.abs(s1_ref[:, 0:8]),
                jnp.abs(s2_ref[:, 0:8]),
                jnp.abs(s3_ref[:, 8:16]),
                jax.nn.relu(g1_ref[:, 0:8]),
            ],
            axis=1,
        )
        outb_ref[...] = jnp.concatenate(
            [
                jax.nn.relu(g2_ref[:, 0:8]),
                jax.nn.relu(g3_ref[:, 8:16]),
                jnp.ones((bn, 1), jnp.float32),
                jnp.zeros((bn, 15), jnp.float32),
            ],
            axis=1,
        )

    return pl.pallas_call(
        body,
        grid=(n // bn,),
        in_specs=[
            pl.BlockSpec((bn, 32), lambda i: (i, 0)),
            pl.BlockSpec((bn, 16), lambda i: (i, 0)),
            pl.BlockSpec((bn, 16), lambda i: (i, 0)),
            pl.BlockSpec((bn, 32), lambda i: (i, 0)),
            pl.BlockSpec((bn, 16), lambda i: (i, 0)),
            pl.BlockSpec((bn, 16), lambda i: (i, 0)),
        ],
        out_specs=[
            pl.BlockSpec((bn, 32), lambda i: (i, 0)),
            pl.BlockSpec((bn, 32), lambda i: (i, 0)),
        ],
        out_shape=[
            jax.ShapeDtypeStruct((n, 32), jnp.float32),
            jax.ShapeDtypeStruct((n, 32), jnp.float32),
        ],
    )(g1, g2, g3, s1, s2, s3)


def _dense_out(n, pA, pB, W_res, b_res):
    do = W_res.shape[1]
    bn = 1000

    def body(pa_ref, pb_ref, w_ref, b_ref, out_ref):
        h48 = jnp.concatenate([pa_ref[...], pb_ref[:, 0:16]], axis=1)
        out_ref[...] = (
            jnp.dot(h48, w_ref[...], preferred_element_type=jnp.float32)
            + pb_ref[:, 16:17] * b_ref[...]
        )

    return pl.pallas_call(
        body,
        grid=(n // bn,),
        in_specs=[
            pl.BlockSpec((bn, 32), lambda i: (i, 0)),
            pl.BlockSpec((bn, 32), lambda i: (i, 0)),
            pl.BlockSpec((48, do), lambda i: (0, 0)),
            pl.BlockSpec((1, do), lambda i: (0, 0)),
        ],
        out_specs=pl.BlockSpec((bn, do), lambda i: (i, 0)),
        out_shape=jax.ShapeDtypeStruct((n, do), jnp.float32),
    )(pA, pB, W_res, b_res)


@functools.lru_cache(maxsize=None)
def _make_chain(n_nodes, nch):
    assert nch % RING == 0
    rpt = n_nodes // N_TILES
    mesh = plsc.VectorSubcoreMesh(core_axis_name="c", subcore_axis_name="s")

    scratch = [
        pltpu.VMEM_SHARED((n_nodes, 32), jnp.float32),
        pltpu.VMEM_SHARED((n_nodes, 16), jnp.float32),
        pltpu.VMEM_SHARED((n_nodes, 16), jnp.float32),
        pltpu.VMEM((nch, CHUNK), jnp.int32),
        pltpu.VMEM((nch, CHUNK), jnp.int32),
    ]
    scratch += [pltpu.VMEM((CHUNK,), jnp.float32) for _ in range(RING)]
    scratch += [pltpu.VMEM((CHUNK, 32), jnp.float32) for _ in range(2 * RING)]
    scratch += [pltpu.VMEM((CHUNK, 16), jnp.float32) for _ in range(4)]
    scratch += [pltpu.SemaphoreType.DMA for _ in range(2 * RING)]

    @functools.partial(
        pl.kernel,
        out_type=[
            jax.ShapeDtypeStruct((n_nodes, 32), jnp.float32),
            jax.ShapeDtypeStruct((n_nodes, 32), jnp.float32),
            jax.ShapeDtypeStruct((n_nodes, 16), jnp.float32),
            jax.ShapeDtypeStruct((n_nodes, 16), jnp.float32),
            jax.ShapeDtypeStruct((n_nodes, 16), jnp.float32),
            jax.ShapeDtypeStruct((n_nodes, 16), jnp.float32),
        ],
        mesh=mesh,
        scratch_types=scratch,
        compiler_params=pltpu.CompilerParams(use_tc_tiling_on_sc=False),
    )
    def chain(hg, hs, wA, wB, src, dst, g1o, s1o, g2o, s2o, g3o, s3o,
              acc1, acc2, acc3, src_all, dst_all, *bufs):
        wbuf = bufs[0:RING]
        buf = bufs[RING:2 * RING]
        obuf = bufs[2 * RING:3 * RING]
        obuf16 = bufs[3 * RING:3 * RING + 2]
        buf16 = bufs[3 * RING + 2:3 * RING + 4]
        gsem = bufs[3 * RING + 4:4 * RING + 4]
        ssem = bufs[4 * RING + 4:5 * RING + 4]

        cid = lax.axis_index("c")
        sid = lax.axis_index("s")
        r0 = sid * rpt
        c0 = sid * nch

        z16 = jnp.zeros((LANES,), jnp.float32)
        for col in range(0, 32, LANES):
            for row in range(CHUNK):
                obuf[0][row, pl.ds(col, LANES)] = z16
        for col in range(0, 16, LANES):
            for row in range(CHUNK):
                obuf16[0][row, pl.ds(col, LANES)] = z16
        off = 0
        while off < rpt:
            size = min(CHUNK, rpt - off)
            sl = pl.ds(r0 + off, size)
            pltpu.sync_copy(obuf[0].at[pl.ds(0, size)], acc1.at[sl])
            pltpu.sync_copy(obuf16[0].at[pl.ds(0, size)], acc2.at[sl])
            pltpu.sync_copy(obuf16[0].at[pl.ds(0, size)], acc3.at[sl])
            off += size

        pltpu.sync_copy(src.at[pl.ds(c0, nch)], src_all)
        pltpu.sync_copy(dst.at[pl.ds(c0, nch)], dst_all)
        plsc.subcore_barrier()

        def issue_w(c, b):
            @pl.when(cid == 0)
            def _():
                pltpu.async_copy(wA.at[c0 + c], wbuf[b], gsem[b])

            @pl.when(cid == 1)
            def _():
                pltpu.async_copy(wB.at[c0 + c], wbuf[b], gsem[b])

        def wait_w(c, b):
            @pl.when(cid == 0)
            def _():
                pltpu.make_async_copy(wA.at[c0 + c], wbuf[b], gsem[b]).wait()

            @pl.when(cid == 1)
            def _():
                pltpu.make_async_copy(wB.at[c0 + c], wbuf[b], gsem[b]).wait()

        def run_pass(ring, kout, col_in, gather_from, acc_out, ib, ob):
            steps = nch // ring

            gA, gB = gather_from

            def issue_gather(c, b):
                @pl.when(cid == 0)
                def _():
                    pltpu.async_copy(gA.at[src_all.at[c]], ib[b], gsem[b])

                @pl.when(cid == 1)
                def _():
                    pltpu.async_copy(gB.at[src_all.at[c]], ib[b], gsem[b])
                issue_w(c, b)

            def wait_gather(c, b):
                @pl.when(cid == 0)
                def _():
                    pltpu.make_async_copy(
                        gA.at[src_all.at[c]], ib[b], gsem[b]).wait()

                @pl.when(cid == 1)
                def _():
                    pltpu.make_async_copy(
                        gB.at[src_all.at[c]], ib[b], gsem[b]).wait()
                wait_w(c, b)

            def drain_scatter(c, b):
                pltpu.make_async_copy(
                    ob[b], acc_out.at[dst_all.at[c]], ssem[b]).wait()

            for b in range(ring):
                issue_gather(b, b)

            def process(js, c, b):
                wait_gather(c, b)

                @pl.when(js > 0)
                def _():
                    drain_scatter(c, b)

                for g in range(CHUNK // LANES):
                    w16 = wbuf[b][pl.ds(g * LANES, LANES)]
                    for e in range(LANES):
                        ec = g * LANES + e
                        for kk in range(kout // LANES):
                            slo = pl.ds(kk * LANES, LANES)
                            sli = pl.ds(col_in + kk * LANES, LANES)
                            ob[b][ec, slo] = ib[b][ec, sli] * w16[e]

                pltpu.async_copy(ob[b], acc_out.at[dst_all.at[c]], ssem[b],
                                 add=True)

                @pl.when(js < steps - 1)
                def _():
                    issue_gather(c + ring, b)

            def body(js, carry):
                for b in range(ring):
                    process(js, ring * js + b, b)
                return carry

            lax.fori_loop(0, steps, body, 0)
            for b in range(ring):
                drain_scatter(b, b)
            plsc.subcore_barrier()

        rsl = pl.ds(r0, rpt)

        def publish(acc_src, oA, oB):
            @pl.when(cid == 0)
            def _():
                pltpu.sync_copy(acc_src.at[rsl], oA.at[rsl])

            @pl.when(cid == 1)
            def _():
                pltpu.sync_copy(acc_src.at[rsl], oB.at[rsl])

            plsc.subcore_barrier()

        run_pass(RING, 32, 0, (hg, hs), acc1, buf, obuf)
        publish(acc1, g1o, s1o)
        run_pass(2, 16, 8, (g1o, s1o), acc2, buf, obuf16)
        publish(acc2, g2o, s2o)
        run_pass(2, 16, 0, (g2o, s2o), acc3, buf16, obuf16)
        publish(acc3, g3o, s3o)

    return chain


@functools.lru_cache(maxsize=None)
def _make_spmm(k, n_nodes, nch):
    assert nch % RING == 0
    steps = nch // RING
    rpt = n_nodes // N_TILES
    mesh = plsc.VectorSubcoreMesh(core_axis_name="c", subcore_axis_name="s")

    scratch = [
        pltpu.VMEM_SHARED((n_nodes, k), jnp.float32),
        pltpu.VMEM((nch, CHUNK), jnp.int32),
        pltpu.VMEM((nch, CHUNK), jnp.int32),
    ]
    scratch += [pltpu.VMEM((CHUNK,), jnp.float32) for _ in range(RING)]
    scratch += [pltpu.VMEM((CHUNK, k), jnp.float32) for _ in range(2 * RING)]
    scratch += [pltpu.SemaphoreType.DMA for _ in range(2 * RING)]

    @functools.partial(
        pl.kernel,
        out_type=[
            jax.ShapeDtypeStruct((n_nodes, k), jnp.float32),
            jax.ShapeDtypeStruct((n_nodes, k), jnp.float32),
        ],
        mesh=mesh,
        scratch_types=scratch,
        compiler_params=pltpu.CompilerParams(use_tc_tiling_on_sc=False),
    )
    def spmm(hA, hB, wA, wB, src, dst, outA, outB, acc, src_all, dst_all,
             *bufs):
        wbuf = bufs[0:RING]
        buf = bufs[RING:2 * RING]
        obuf = bufs[2 * RING:3 * RING]
        gsem = bufs[3 * RING:4 * RING]
        ssem = bufs[4 * RING:5 * RING]

        cid = lax.axis_index("c")
        sid = lax.axis_index("s")
        r0 = sid * rpt
        c0 = sid * nch

        for col in range(0, k, LANES):
            z16 = jnp.zeros((LANES,), jnp.float32)
            for row in range(CHUNK):
                obuf[0][row, pl.ds(col, LANES)] = z16
        off = 0
        while off < rpt:
            size = min(CHUNK, rpt - off)
            pltpu.sync_copy(obuf[0].at[pl.ds(0, size)],
                            acc.at[pl.ds(r0 + off, size)])
            off += size

        pltpu.sync_copy(src.at[pl.ds(c0, nch)], src_all)
        pltpu.sync_copy(dst.at[pl.ds(c0, nch)], dst_all)
        plsc.subcore_barrier()

        def issue_gather(c, b):
            @pl.when(cid == 0)
            def _():
                pltpu.async_copy(hA.at[src_all.at[c]], buf[b], gsem[b])
                pltpu.async_copy(wA.at[c0 + c], wbuf[b], gsem[b])

            @pl.when(cid == 1)
            def _():
                pltpu.async_copy(hB.at[src_all.at[c]], buf[b], gsem[b])
                pltpu.async_copy(wB.at[c0 + c], wbuf[b], gsem[b])

        def wait_gather(c, b):
            @pl.when(cid == 0)
            def _():
                pltpu.make_async_copy(
                    hA.at[src_all.at[c]], buf[b], gsem[b]).wait()
                pltpu.make_async_copy(wA.at[c0 + c], wbuf[b], gsem[b]).wait()

            @pl.when(cid == 1)
            def _():
                pltpu.make_async_copy(
                    hB.at[src_all.at[c]], buf[b], gsem[b]).wait()
                pltpu.make_async_copy(wB.at[c0 + c], wbuf[b], gsem[b]).wait()

        def drain_scatter(c, b):
            pltpu.make_async_copy(
                obuf[b], acc.at[dst_all.at[c]], ssem[b]).wait()

        for b in range(RING):
            issue_gather(b, b)

        def process(js, c, b):
            wait_gather(c, b)

            @pl.when(js > 0)
            def _():
                drain_scatter(c, b)

            for g in range(CHUNK // LANES):
                w16 = wbuf[b][pl.ds(g * LANES, LANES)]
                for e in range(LANES):
                    ec = g * LANES + e
                    for kk in range(k // LANES):
                        sl = pl.ds(kk * LANES, LANES)
                        obuf[b][ec, sl] = buf[b][ec, sl] * w16[e]

            pltpu.async_copy(obuf[b], acc.at[dst_all.at[c]], ssem[b],
                             add=True)

            @pl.when(js < steps - 1)
            def _():
                issue_gather(c + RING, b)

        def body(js, carry):
            for b in range(RING):
                process(js, RING * js + b, b)
            return carry

        lax.fori_loop(0, steps, body, 0)
        for b in range(RING):
            drain_scatter(b, b)
        plsc.subcore_barrier()

        @pl.when(cid == 0)
        def _():
            pltpu.sync_copy(acc.at[pl.ds(r0, rpt)], outA.at[pl.ds(r0, rpt)])

        @pl.when(cid == 1)
        def _():
            pltpu.sync_copy(acc.at[pl.ds(r0, rpt)], outB.at[pl.ds(r0, rpt)])

    return spmm


def kernel(x, edge_index, gcn_weight, sct_weight, res_weight,
           W_hyb, b_hyb, W_res, b_res):
    n = x.shape[0]
    e = edge_index.shape[1]

    nch = -(-e // (N_TILES * CHUNK))
    nch = -(-nch // RING) * RING
    pad = nch * N_TILES * CHUNK - e
    zi = jnp.zeros((pad,), jnp.int32)
    zf = jnp.zeros((pad,), jnp.float32)
    src = jnp.concatenate([edge_index[0], zi]).reshape(-1, CHUNK)
    dst = jnp.concatenate([edge_index[1], zi]).reshape(-1, CHUNK)
    wg = jnp.concatenate([gcn_weight, zf]).reshape(-1, CHUNK)
    ws = jnp.concatenate([sct_weight, zf]).reshape(-1, CHUNK)
    wr = jnp.concatenate([res_weight, zf]).reshape(-1, CHUNK)

    d_in = x.shape[1]
    z8 = jnp.zeros((d_in, 8), jnp.float32)
    W_all = jnp.concatenate(
        [W_hyb[3], W_hyb[4], W_hyb[5], z8,
         W_hyb[0], W_hyb[1], W_hyb[2], z8], axis=1)
    zb8 = jnp.zeros((8,), jnp.float32)
    b_all = jnp.concatenate(
        [b_hyb[3], b_hyb[4], b_hyb[5], zb8,
         b_hyb[0], b_hyb[1], b_hyb[2], zb8]).reshape(1, 64)

    hg, hs = _dense_in(x, W_all, b_all)

    npad = -(-n // (N_TILES * 8)) * (N_TILES * 8)

    g1, s1, g2, s2, g3, s3 = _make_chain(npad, nch)(
        hg, hs, wg, ws, src, dst)

    hA, hB = _dense_mid(g1, g2, g3, s1, s2, s3)

    pA, pB = _make_spmm(32, npad, nch)(hA, hB, wr, wr, src, dst)
    return _dense_out(n, pA, pB, W_res, b_res.reshape(1, -1))


# speedup vs baseline: 1.0213x; 1.0213x over previous
import functools

import jax
import jax.numpy as jnp
from jax import lax
from jax.experimental import pallas as pl
from jax.experimental.pallas import tpu as pltpu
from jax.experimental.pallas import tpu_sc as plsc

CHUNK = 128
N_TILES = 16
LANES = 16
RING = 4


def _dense_in(x, W_all, b_all):
    n, d = x.shape
    ko = W_all.shape[1]
    bn = 1000

    def body(x_ref, w_ref, b_ref, outa_ref, outb_ref):
        h = jnp.dot(x_ref[...], w_ref[...],
                    preferred_element_type=jnp.float32) + b_ref[...]
        outa_ref[...] = h[:, : ko // 2]
        outb_ref[...] = h[:, ko // 2:]

    return pl.pallas_call(
        body,
        grid=(n // bn,),
        in_specs=[
            pl.BlockSpec((bn, d), lambda i: (i, 0)),
            pl.BlockSpec((d, ko), lambda i: (0, 0)),
            pl.BlockSpec((1, ko), lambda i: (0, 0)),
        ],
        out_specs=[
            pl.BlockSpec((bn, ko // 2), lambda i: (i, 0)),
            pl.BlockSpec((bn, ko // 2), lambda i: (i, 0)),
        ],
        out_shape=[
            jax.ShapeDtypeStruct((n, ko // 2), jnp.float32),
            jax.ShapeDtypeStruct((n, ko // 2), jnp.float32),
        ],
    )(x, W_all, b_all)


def _dense_mid(g1, g2, g3, s1, s2, s3):
    n = g1.shape[0]
    bn = n // 16

    def body(g1_ref, g2_ref, g3_ref, s1_ref, s2_ref, s3_ref,
             outa_ref, outb_ref):
        outa_ref[...] = jnp.concatenate(
            [
                jnp.abs(s1_ref[:, 0:8]),
                jnp.abs(s2_ref[:, 0:8]),
                jnp.abs(s3_ref[:, 8:16]),
                jax.nn.relu(g1_ref[:, 0:8]),
            ],
            axis=1,
        )
        outb_ref[...] = jnp.concatenate(
            [
                jax.nn.relu(g2_ref[:, 0:8]),
                jax.nn.relu(g3_ref[:, 8:16]),
                jnp.ones((bn, 1), jnp.float32),
                jnp.zeros((bn, 15), jnp.float32),
            ],
            axis=1,
        )

    return pl.pallas_call(
        body,
        grid=(n // bn,),
        in_specs=[
            pl.BlockSpec((bn, 24), lambda i: (i, 0)),
            pl.BlockSpec((bn, 16), lambda i: (i, 0)),
            pl.BlockSpec((bn, 16), lambda i: (i, 0)),
            pl.BlockSpec((bn, 24), lambda i: (i, 0)),
            pl.BlockSpec((bn, 16), lambda i: (i, 0)),
            pl.BlockSpec((bn, 16), lambda i: (i, 0)),
        ],
        out_specs=[
            pl.BlockSpec((bn, 32), lambda i: (i, 0)),
            pl.BlockSpec((bn, 32), lambda i: (i, 0)),
        ],
        out_shape=[
            jax.ShapeDtypeStruct((n, 32), jnp.float32),
            jax.ShapeDtypeStruct((n, 32), jnp.float32),
        ],
    )(g1, g2, g3, s1, s2, s3)


def _dense_out(n, pA, pB, W_res, b_res):
    do = W_res.shape[1]
    bn = 1000

    def body(pa_ref, pb_ref, w_ref, b_ref, out_ref):
        h48 = jnp.concatenate([pa_ref[...], pb_ref[:, 0:16]], axis=1)
        out_ref[...] = (
            jnp.dot(h48, w_ref[...], preferred_element_type=jnp.float32)
            + pb_ref[:, 16:17] * b_ref[...]
        )

    return pl.pallas_call(
        body,
        grid=(n // bn,),
        in_specs=[
            pl.BlockSpec((bn, 32), lambda i: (i, 0)),
            pl.BlockSpec((bn, 32), lambda i: (i, 0)),
            pl.BlockSpec((48, do), lambda i: (0, 0)),
            pl.BlockSpec((1, do), lambda i: (0, 0)),
        ],
        out_specs=pl.BlockSpec((bn, do), lambda i: (i, 0)),
        out_shape=jax.ShapeDtypeStruct((n, do), jnp.float32),
    )(pA, pB, W_res, b_res)


@functools.lru_cache(maxsize=None)
def _make_spmm(kin, kout, col_in, n_nodes, nch):
    assert nch % RING == 0
    steps = nch // RING
    rpt = n_nodes // N_TILES
    mesh = plsc.VectorSubcoreMesh(core_axis_name="c", subcore_axis_name="s")

    scratch = [
        pltpu.VMEM_SHARED((n_nodes, kout), jnp.float32),
        pltpu.VMEM((nch, CHUNK), jnp.int32),
        pltpu.VMEM((nch, CHUNK), jnp.int32),
    ]
    scratch += [pltpu.VMEM((CHUNK,), jnp.float32) for _ in range(RING)]
    scratch += [pltpu.VMEM((CHUNK, kin), jnp.float32) for _ in range(RING)]
    scratch += [pltpu.VMEM((CHUNK, kout), jnp.float32) for _ in range(RING)]
    scratch += [pltpu.SemaphoreType.DMA for _ in range(2 * RING)]

    offs = list(range(0, kout - LANES + 1, LANES))
    if kout % LANES:
        offs.append(kout - LANES)
    assert kout % 8 == 0 and col_in % 8 == 0

    @functools.partial(
        pl.kernel,
        out_type=[
            jax.ShapeDtypeStruct((n_nodes, kout), jnp.float32),
            jax.ShapeDtypeStruct((n_nodes, kout), jnp.float32),
        ],
        mesh=mesh,
        scratch_types=scratch,
        compiler_params=pltpu.CompilerParams(use_tc_tiling_on_sc=False),
    )
    def spmm(hA, hB, wA, wB, src, dst, outA, outB, acc, src_all, dst_all,
             *bufs):
        wbuf = bufs[0:RING]
        buf = bufs[RING:2 * RING]
        obuf = bufs[2 * RING:3 * RING]
        gsem = bufs[3 * RING:4 * RING]
        ssem = bufs[4 * RING:5 * RING]

        cid = lax.axis_index("c")
        sid = lax.axis_index("s")
        r0 = sid * rpt
        c0 = sid * nch

        z16 = jnp.zeros((LANES,), jnp.float32)
        for col in offs:
            for row in range(CHUNK):
                obuf[0][row, pl.ds(col, LANES)] = z16
        off = 0
        while off < rpt:
            size = min(CHUNK, rpt - off)
            pltpu.sync_copy(obuf[0].at[pl.ds(0, size)],
                            acc.at[pl.ds(r0 + off, size)])
            off += size

        pltpu.sync_copy(src.at[pl.ds(c0, nch)], src_all)
        pltpu.sync_copy(dst.at[pl.ds(c0, nch)], dst_all)
        plsc.subcore_barrier()

        def issue_gather(c, b):
            @pl.when(cid == 0)
            def _():
                pltpu.async_copy(hA.at[src_all.at[c]], buf[b], gsem[b])
                pltpu.async_copy(wA.at[c0 + c], wbuf[b], gsem[b])

            @pl.when(cid == 1)
            def _():
                pltpu.async_copy(hB.at[src_all.at[c]], buf[b], gsem[b])
                pltpu.async_copy(wB.at[c0 + c], wbuf[b], gsem[b])

        def wait_gather(c, b):
            @pl.when(cid == 0)
            def _():
                pltpu.make_async_copy(
                    hA.at[src_all.at[c]], buf[b], gsem[b]).wait()
                pltpu.make_async_copy(wA.at[c0 + c], wbuf[b], gsem[b]).wait()

            @pl.when(cid == 1)
            def _():
                pltpu.make_async_copy(
                    hB.at[src_all.at[c]], buf[b], gsem[b]).wait()
                pltpu.make_async_copy(wB.at[c0 + c], wbuf[b], gsem[b]).wait()

        def drain_scatter(c, b):
            pltpu.make_async_copy(
                obuf[b], acc.at[dst_all.at[c]], ssem[b]).wait()

        for b in range(RING):
            issue_gather(b, b)

        def process(js, c, b):
            wait_gather(c, b)

            @pl.when(js > 0)
            def _():
                drain_scatter(c, b)

            for g in range(CHUNK // LANES):
                w16 = wbuf[b][pl.ds(g * LANES, LANES)]
                for e in range(LANES):
                    ec = g * LANES + e
                    for off in offs:
                        slo = pl.ds(off, LANES)
                        sli = pl.ds(col_in + off, LANES)
                        obuf[b][ec, slo] = buf[b][ec, sli] * w16[e]

            pltpu.async_copy(obuf[b], acc.at[dst_all.at[c]], ssem[b],
                             add=True)

            @pl.when(js < steps - 1)
            def _():
                issue_gather(c + RING, b)

        def body(js, carry):
            for b in range(RING):
                process(js, RING * js + b, b)
            return carry

        lax.fori_loop(0, steps, body, 0)
        for b in range(RING):
            drain_scatter(b, b)
        plsc.subcore_barrier()

        @pl.when(cid == 0)
        def _():
            pltpu.sync_copy(acc.at[pl.ds(r0, rpt)], outA.at[pl.ds(r0, rpt)])

        @pl.when(cid == 1)
        def _():
            pltpu.sync_copy(acc.at[pl.ds(r0, rpt)], outB.at[pl.ds(r0, rpt)])

    return spmm


def kernel(x, edge_index, gcn_weight, sct_weight, res_weight,
           W_hyb, b_hyb, W_res, b_res):
    n = x.shape[0]
    e = edge_index.shape[1]

    nch = -(-e // (N_TILES * CHUNK))
    nch = -(-nch // RING) * RING
    pad = nch * N_TILES * CHUNK - e
    zi = jnp.zeros((pad,), jnp.int32)
    zf = jnp.zeros((pad,), jnp.float32)
    src = jnp.concatenate([edge_index[0], zi]).reshape(-1, CHUNK)
    dst = jnp.concatenate([edge_index[1], zi]).reshape(-1, CHUNK)
    wg = jnp.concatenate([gcn_weight, zf]).reshape(-1, CHUNK)
    ws = jnp.concatenate([sct_weight, zf]).reshape(-1, CHUNK)
    wr = jnp.concatenate([res_weight, zf]).reshape(-1, CHUNK)

    W_all = jnp.concatenate(
        [W_hyb[3], W_hyb[4], W_hyb[5],
         W_hyb[0], W_hyb[1], W_hyb[2]], axis=1)
    b_all = jnp.concatenate(
        [b_hyb[3], b_hyb[4], b_hyb[5],
         b_hyb[0], b_hyb[1], b_hyb[2]]).reshape(1, 48)

    hg, hs = _dense_in(x, W_all, b_all)

    npad = -(-n // (N_TILES * 8)) * (N_TILES * 8)

    g1, s1 = _make_spmm(24, 24, 0, npad, nch)(hg, hs, wg, ws, src, dst)
    g2, s2 = _make_spmm(24, 16, 8, npad, nch)(g1, s1, wg, ws, src, dst)
    g3, s3 = _make_spmm(16, 16, 0, npad, nch)(g2, s2, wg, ws, src, dst)

    hA, hB = _dense_mid(g1, g2, g3, s1, s2, s3)

    pA, pB = _make_spmm(32, 32, 0, npad, nch)(hA, hB, wr, wr, src, dst)
    return _dense_out(n, pA, pB, W_res, b_res.reshape(1, -1))


# speedup vs baseline: 1.0773x; 1.0549x over previous
import functools

import jax
import jax.numpy as jnp
from jax import lax
from jax.experimental import pallas as pl
from jax.experimental.pallas import tpu as pltpu
from jax.experimental.pallas import tpu_sc as plsc

CHUNK = 128
N_TILES = 16
LANES = 16
RING = 4


def _dense_in(x, W_all, b_all):
    n, d = x.shape
    ko = W_all.shape[1]
    bn = 1000

    def body(x_ref, w_ref, b_ref, outa_ref, outb_ref):
        h = jnp.dot(x_ref[...], w_ref[...],
                    preferred_element_type=jnp.float32) + b_ref[...]
        outa_ref[...] = h[:, : ko // 2]
        outb_ref[...] = h[:, ko // 2:]

    return pl.pallas_call(
        body,
        grid=(n // bn,),
        in_specs=[
            pl.BlockSpec((bn, d), lambda i: (i, 0)),
            pl.BlockSpec((d, ko), lambda i: (0, 0)),
            pl.BlockSpec((1, ko), lambda i: (0, 0)),
        ],
        out_specs=[
            pl.BlockSpec((bn, ko // 2), lambda i: (i, 0)),
            pl.BlockSpec((bn, ko // 2), lambda i: (i, 0)),
        ],
        out_shape=[
            jax.ShapeDtypeStruct((n, ko // 2), jnp.float32),
            jax.ShapeDtypeStruct((n, ko // 2), jnp.float32),
        ],
    )(x, W_all, b_all)


def _dense_mid(g1, g2, g3, s1, s2, s3):
    n = g1.shape[0]
    bn = n // 16

    def body(g1_ref, g2_ref, g3_ref, s1_ref, s2_ref, s3_ref,
             outa_ref, outb_ref):
        outa_ref[...] = jnp.concatenate(
            [
                jnp.abs(s1_ref[:, 0:8]),
                jnp.abs(s2_ref[:, 0:8]),
                jnp.abs(s3_ref[:, 8:16]),
                jax.nn.relu(g1_ref[:, 0:8]),
            ],
            axis=1,
        )
        outb_ref[...] = jnp.concatenate(
            [
                jax.nn.relu(g2_ref[:, 0:8]),
                jax.nn.relu(g3_ref[:, 8:16]),
                jnp.ones((bn, 1), jnp.float32),
                jnp.zeros((bn, 15), jnp.float32),
            ],
            axis=1,
        )

    return pl.pallas_call(
        body,
        grid=(n // bn,),
        in_specs=[
            pl.BlockSpec((bn, 32), lambda i: (i, 0)),
            pl.BlockSpec((bn, 16), lambda i: (i, 0)),
            pl.BlockSpec((bn, 16), lambda i: (i, 0)),
            pl.BlockSpec((bn, 32), lambda i: (i, 0)),
            pl.BlockSpec((bn, 16), lambda i: (i, 0)),
            pl.BlockSpec((bn, 16), lambda i: (i, 0)),
        ],
        out_specs=[
            pl.BlockSpec((bn, 32), lambda i: (i, 0)),
            pl.BlockSpec((bn, 32), lambda i: (i, 0)),
        ],
        out_shape=[
            jax.ShapeDtypeStruct((n, 32), jnp.float32),
            jax.ShapeDtypeStruct((n, 32), jnp.float32),
        ],
    )(g1, g2, g3, s1, s2, s3)


def _dense_out(n, pA, pB, W_res, b_res):
    do = W_res.shape[1]
    bn = 1000

    def body(pa_ref, pb_ref, w_ref, b_ref, out_ref):
        h48 = jnp.concatenate([pa_ref[...], pb_ref[:, 0:16]], axis=1)
        out_ref[...] = (
            jnp.dot(h48, w_ref[...], preferred_element_type=jnp.float32)
            + pb_ref[:, 16:17] * b_ref[...]
        )

    return pl.pallas_call(
        body,
        grid=(n // bn,),
        in_specs=[
            pl.BlockSpec((bn, 32), lambda i: (i, 0)),
            pl.BlockSpec((bn, 32), lambda i: (i, 0)),
            pl.BlockSpec((48, do), lambda i: (0, 0)),
            pl.BlockSpec((1, do), lambda i: (0, 0)),
        ],
        out_specs=pl.BlockSpec((bn, do), lambda i: (i, 0)),
        out_shape=jax.ShapeDtypeStruct((n, do), jnp.float32),
    )(pA, pB, W_res, b_res)


@functools.lru_cache(maxsize=None)
def _make_spmm(kin, kout, col_in, n_nodes, nch):
    assert nch % RING == 0
    steps = nch // RING
    rpt = n_nodes // N_TILES
    mesh = plsc.VectorSubcoreMesh(core_axis_name="c", subcore_axis_name="s")

    scratch = [
        pltpu.VMEM_SHARED((n_nodes, kout), jnp.float32),
        pltpu.VMEM((nch, CHUNK), jnp.int32),
        pltpu.VMEM((nch, CHUNK), jnp.int32),
    ]
    scratch += [pltpu.VMEM((CHUNK,), jnp.float32) for _ in range(RING)]
    scratch += [pltpu.VMEM((CHUNK, kin), jnp.float32) for _ in range(RING)]
    scratch += [pltpu.VMEM((CHUNK, kout), jnp.float32) for _ in range(RING)]
    scratch += [pltpu.SemaphoreType.DMA for _ in range(2 * RING)]

    offs = list(range(0, kout - LANES + 1, LANES))
    if kout % LANES:
        offs.append(kout - LANES)
    assert kout % 8 == 0 and col_in % 8 == 0

    @functools.partial(
        pl.kernel,
        out_type=[
            jax.ShapeDtypeStruct((n_nodes, kout), jnp.float32),
            jax.ShapeDtypeStruct((n_nodes, kout), jnp.float32),
        ],
        mesh=mesh,
        scratch_types=scratch,
        compiler_params=pltpu.CompilerParams(use_tc_tiling_on_sc=False),
    )
    def spmm(hA, hB, wA, wB, src, dst, outA, outB, acc, src_all, dst_all,
             *bufs):
        wbuf = bufs[0:RING]
        buf = bufs[RING:2 * RING]
        obuf = bufs[2 * RING:3 * RING]
        gsem = bufs[3 * RING:4 * RING]
        ssem = bufs[4 * RING:5 * RING]

        cid = lax.axis_index("c")
        sid = lax.axis_index("s")
        r0 = sid * rpt
        c0 = sid * nch

        z16 = jnp.zeros((LANES,), jnp.float32)
        for col in offs:
            for row in range(CHUNK):
                obuf[0][row, pl.ds(col, LANES)] = z16
        off = 0
        while off < rpt:
            size = min(CHUNK, rpt - off)
            pltpu.sync_copy(obuf[0].at[pl.ds(0, size)],
                            acc.at[pl.ds(r0 + off, size)])
            off += size

        pltpu.sync_copy(src.at[pl.ds(c0, nch)], src_all)
        pltpu.sync_copy(dst.at[pl.ds(c0, nch)], dst_all)
        plsc.subcore_barrier()

        def issue_gather(c, b):
            @pl.when(cid == 0)
            def _():
                pltpu.async_copy(hA.at[src_all.at[c]], buf[b], gsem[b])
                pltpu.async_copy(wA.at[c0 + c], wbuf[b], gsem[b])

            @pl.when(cid == 1)
            def _():
                pltpu.async_copy(hB.at[src_all.at[c]], buf[b], gsem[b])
                pltpu.async_copy(wB.at[c0 + c], wbuf[b], gsem[b])

        def wait_gather(c, b):
            @pl.when(cid == 0)
            def _():
                pltpu.make_async_copy(
                    hA.at[src_all.at[c]], buf[b], gsem[b]).wait()
                pltpu.make_async_copy(wA.at[c0 + c], wbuf[b], gsem[b]).wait()

            @pl.when(cid == 1)
            def _():
                pltpu.make_async_copy(
                    hB.at[src_all.at[c]], buf[b], gsem[b]).wait()
                pltpu.make_async_copy(wB.at[c0 + c], wbuf[b], gsem[b]).wait()

        def drain_scatter(c, b):
            pltpu.make_async_copy(
                obuf[b], acc.at[dst_all.at[c]], ssem[b]).wait()

        for b in range(RING):
            issue_gather(b, b)

        def process(js, c, b):
            wait_gather(c, b)

            @pl.when(js > 0)
            def _():
                drain_scatter(c, b)

            for g in range(CHUNK // LANES):
                w16 = wbuf[b][pl.ds(g * LANES, LANES)]
                for e in range(LANES):
                    ec = g * LANES + e
                    for off in offs:
                        slo = pl.ds(off, LANES)
                        sli = pl.ds(col_in + off, LANES)
                        obuf[b][ec, slo] = buf[b][ec, sli] * w16[e]

            pltpu.async_copy(obuf[b], acc.at[dst_all.at[c]], ssem[b],
                             add=True)

            @pl.when(js < steps - 1)
            def _():
                issue_gather(c + RING, b)

        def body(js, carry):
            for b in range(RING):
                process(js, RING * js + b, b)
            return carry

        lax.fori_loop(0, steps, body, 0)
        for b in range(RING):
            drain_scatter(b, b)
        plsc.subcore_barrier()

        @pl.when(cid == 0)
        def _():
            pltpu.sync_copy(acc.at[pl.ds(r0, rpt)], outA.at[pl.ds(r0, rpt)])

        @pl.when(cid == 1)
        def _():
            pltpu.sync_copy(acc.at[pl.ds(r0, rpt)], outB.at[pl.ds(r0, rpt)])

    return spmm


def kernel(x, edge_index, gcn_weight, sct_weight, res_weight,
           W_hyb, b_hyb, W_res, b_res):
    n = x.shape[0]
    e = edge_index.shape[1]

    nch = -(-e // (N_TILES * CHUNK))
    nch = -(-nch // RING) * RING
    pad = nch * N_TILES * CHUNK - e
    zi = jnp.zeros((pad,), jnp.int32)
    zf = jnp.zeros((pad,), jnp.float32)
    src = jnp.concatenate([edge_index[0], zi]).reshape(-1, CHUNK)
    dst = jnp.concatenate([edge_index[1], zi]).reshape(-1, CHUNK)
    wg = jnp.concatenate([gcn_weight, zf]).reshape(-1, CHUNK)
    ws = jnp.concatenate([sct_weight, zf]).reshape(-1, CHUNK)
    wr = jnp.concatenate([res_weight, zf]).reshape(-1, CHUNK)

    d_in = x.shape[1]
    z8 = jnp.zeros((d_in, 8), jnp.float32)
    W_all = jnp.concatenate(
        [W_hyb[3], W_hyb[4], W_hyb[5], z8,
         W_hyb[0], W_hyb[1], W_hyb[2], z8], axis=1)
    zb8 = jnp.zeros((8,), jnp.float32)
    b_all = jnp.concatenate(
        [b_hyb[3], b_hyb[4], b_hyb[5], zb8,
         b_hyb[0], b_hyb[1], b_hyb[2], zb8]).reshape(1, 64)

    hg, hs = _dense_in(x, W_all, b_all)

    npad = -(-n // (N_TILES * 8)) * (N_TILES * 8)

    g1, s1 = _make_spmm(32, 32, 0, npad, nch)(hg, hs, wg, ws, src, dst)
    g2, s2 = _make_spmm(16, 16, 0, npad, nch)(
        g1[:, 8:24], s1[:, 8:24], wg, ws, src, dst)
    g3, s3 = _make_spmm(16, 16, 0, npad, nch)(g2, s2, wg, ws, src, dst)

    hA, hB = _dense_mid(g1, g2, g3, s1, s2, s3)

    pA, pB = _make_spmm(32, 32, 0, npad, nch)(hA, hB, wr, wr, src, dst)
    return _dense_out(n, pA, pB, W_res, b_res.reshape(1, -1))


# speedup vs baseline: 1.0780x; 1.0006x over previous
import functools

import jax
import jax.numpy as jnp
from jax import lax
from jax.experimental import pallas as pl
from jax.experimental.pallas import tpu as pltpu
from jax.experimental.pallas import tpu_sc as plsc

CHUNK = 128
N_TILES = 16
LANES = 16
RING = 4


def _dense_in(x, W_all, b_all):
    n, d = x.shape
    ko = W_all.shape[1]
    bn = 1000

    def body(x_ref, w_ref, b_ref, outa_ref, outb_ref):
        h = jnp.dot(x_ref[...], w_ref[...],
                    preferred_element_type=jnp.float32) + b_ref[...]
        outa_ref[...] = h[:, : ko // 2]
        outb_ref[...] = h[:, ko // 2:]

    return pl.pallas_call(
        body,
        grid=(n // bn,),
        in_specs=[
            pl.BlockSpec((bn, d), lambda i: (i, 0)),
            pl.BlockSpec((d, ko), lambda i: (0, 0)),
            pl.BlockSpec((1, ko), lambda i: (0, 0)),
        ],
        out_specs=[
            pl.BlockSpec((bn, ko // 2), lambda i: (i, 0)),
            pl.BlockSpec((bn, ko // 2), lambda i: (i, 0)),
        ],
        out_shape=[
            jax.ShapeDtypeStruct((n, ko // 2), jnp.float32),
            jax.ShapeDtypeStruct((n, ko // 2), jnp.float32),
        ],
    )(x, W_all, b_all)


def _dense_mid(g1, g2, g3, s1, s2, s3):
    n = g1.shape[0]
    bn = n // 16

    def body(g1_ref, g2_ref, g3_ref, s1_ref, s2_ref, s3_ref,
             outa_ref, outb_ref):
        outa_ref[...] = jnp.concatenate(
            [
                jnp.abs(s1_ref[:, 0:8]),
                jnp.abs(s2_ref[:, 0:8]),
                jnp.abs(s3_ref[:, 8:16]),
                jax.nn.relu(g1_ref[:, 0:8]),
            ],
            axis=1,
        )
        outb_ref[...] = jnp.concatenate(
            [
                jax.nn.relu(g2_ref[:, 0:8]),
                jax.nn.relu(g3_ref[:, 8:16]),
                jnp.ones((bn, 1), jnp.float32),
                jnp.zeros((bn, 15), jnp.float32),
            ],
            axis=1,
        )

    return pl.pallas_call(
        body,
        grid=(n // bn,),
        in_specs=[
            pl.BlockSpec((bn, 32), lambda i: (i, 0)),
            pl.BlockSpec((bn, 16), lambda i: (i, 0)),
            pl.BlockSpec((bn, 16), lambda i: (i, 0)),
            pl.BlockSpec((bn, 32), lambda i: (i, 0)),
            pl.BlockSpec((bn, 16), lambda i: (i, 0)),
            pl.BlockSpec((bn, 16), lambda i: (i, 0)),
        ],
        out_specs=[
            pl.BlockSpec((bn, 32), lambda i: (i, 0)),
            pl.BlockSpec((bn, 32), lambda i: (i, 0)),
        ],
        out_shape=[
            jax.ShapeDtypeStruct((n, 32), jnp.float32),
            jax.ShapeDtypeStruct((n, 32), jnp.float32),
        ],
    )(g1, g2, g3, s1, s2, s3)


def _dense_out(n, pA, pB, W_res, b_res):
    do = W_res.shape[1]
    bn = 1000

    def body(pa_ref, pb_ref, w_ref, b_ref, out_ref):
        h48 = jnp.concatenate([pa_ref[...], pb_ref[:, 0:16]], axis=1)
        out_ref[...] = (
            jnp.dot(h48, w_ref[...], preferred_element_type=jnp.float32)
            + pb_ref[:, 16:17] * b_ref[...]
        )

    return pl.pallas_call(
        body,
        grid=(n // bn,),
        in_specs=[
            pl.BlockSpec((bn, 32), lambda i: (i, 0)),
            pl.BlockSpec((bn, 32), lambda i: (i, 0)),
            pl.BlockSpec((48, do), lambda i: (0, 0)),
            pl.BlockSpec((1, do), lambda i: (0, 0)),
        ],
        out_specs=pl.BlockSpec((bn, do), lambda i: (i, 0)),
        out_shape=jax.ShapeDtypeStruct((n, do), jnp.float32),
    )(pA, pB, W_res, b_res)


@functools.lru_cache(maxsize=None)
def _make_spmm(kin, kout, col_in, n_nodes, nch, ring=RING):
    assert nch % ring == 0
    steps = nch // ring
    rpt = n_nodes // N_TILES
    mesh = plsc.VectorSubcoreMesh(core_axis_name="c", subcore_axis_name="s")

    scratch = [
        pltpu.VMEM_SHARED((n_nodes, kout), jnp.float32),
        pltpu.VMEM((nch, CHUNK), jnp.int32),
        pltpu.VMEM((nch, CHUNK), jnp.int32),
    ]
    scratch += [pltpu.VMEM((CHUNK,), jnp.float32) for _ in range(ring)]
    scratch += [pltpu.VMEM((CHUNK, kin), jnp.float32) for _ in range(ring)]
    scratch += [pltpu.VMEM((CHUNK, kout), jnp.float32) for _ in range(ring)]
    scratch += [pltpu.SemaphoreType.DMA for _ in range(2 * ring)]

    offs = list(range(0, kout - LANES + 1, LANES))
    if kout % LANES:
        offs.append(kout - LANES)
    assert kout % 8 == 0 and col_in % 8 == 0

    @functools.partial(
        pl.kernel,
        out_type=[
            jax.ShapeDtypeStruct((n_nodes, kout), jnp.float32),
            jax.ShapeDtypeStruct((n_nodes, kout), jnp.float32),
        ],
        mesh=mesh,
        scratch_types=scratch,
        compiler_params=pltpu.CompilerParams(use_tc_tiling_on_sc=False),
    )
    def spmm(hA, hB, wA, wB, src, dst, outA, outB, acc, src_all, dst_all,
             *bufs):
        wbuf = bufs[0:ring]
        buf = bufs[ring:2 * ring]
        obuf = bufs[2 * ring:3 * ring]
        gsem = bufs[3 * ring:4 * ring]
        ssem = bufs[4 * ring:5 * ring]

        cid = lax.axis_index("c")
        sid = lax.axis_index("s")
        r0 = sid * rpt
        c0 = sid * nch

        z16 = jnp.zeros((LANES,), jnp.float32)
        for col in offs:
            for row in range(CHUNK):
                obuf[0][row, pl.ds(col, LANES)] = z16
        off = 0
        while off < rpt:
            size = min(CHUNK, rpt - off)
            pltpu.sync_copy(obuf[0].at[pl.ds(0, size)],
                            acc.at[pl.ds(r0 + off, size)])
            off += size

        pltpu.sync_copy(src.at[pl.ds(c0, nch)], src_all)
        pltpu.sync_copy(dst.at[pl.ds(c0, nch)], dst_all)
        plsc.subcore_barrier()

        def issue_gather(c, b):
            @pl.when(cid == 0)
            def _():
                pltpu.async_copy(hA.at[src_all.at[c]], buf[b], gsem[b])
                pltpu.async_copy(wA.at[c0 + c], wbuf[b], gsem[b])

            @pl.when(cid == 1)
            def _():
                pltpu.async_copy(hB.at[src_all.at[c]], buf[b], gsem[b])
                pltpu.async_copy(wB.at[c0 + c], wbuf[b], gsem[b])

        def wait_gather(c, b):
            @pl.when(cid == 0)
            def _():
                pltpu.make_async_copy(
                    hA.at[src_all.at[c]], buf[b], gsem[b]).wait()
                pltpu.make_async_copy(wA.at[c0 + c], wbuf[b], gsem[b]).wait()

            @pl.when(cid == 1)
            def _():
                pltpu.make_async_copy(
                    hB.at[src_all.at[c]], buf[b], gsem[b]).wait()
                pltpu.make_async_copy(wB.at[c0 + c], wbuf[b], gsem[b]).wait()

        def drain_scatter(c, b):
            pltpu.make_async_copy(
                obuf[b], acc.at[dst_all.at[c]], ssem[b]).wait()

        for b in range(ring):
            issue_gather(b, b)

        def process(js, c, b):
            wait_gather(c, b)

            @pl.when(js > 0)
            def _():
                drain_scatter(c, b)

            for g in range(CHUNK // LANES):
                w16 = wbuf[b][pl.ds(g * LANES, LANES)]
                for e in range(LANES):
                    ec = g * LANES + e
                    for off in offs:
                        slo = pl.ds(off, LANES)
                        sli = pl.ds(col_in + off, LANES)
                        obuf[b][ec, slo] = buf[b][ec, sli] * w16[e]

            pltpu.async_copy(obuf[b], acc.at[dst_all.at[c]], ssem[b],
                             add=True)

            @pl.when(js < steps - 1)
            def _():
                issue_gather(c + RING, b)

        def body(js, carry):
            for b in range(ring):
                process(js, ring * js + b, b)
            return carry

        lax.fori_loop(0, steps, body, 0)
        for b in range(ring):
            drain_scatter(b, b)
        plsc.subcore_barrier()

        @pl.when(cid == 0)
        def _():
            pltpu.sync_copy(acc.at[pl.ds(r0, rpt)], outA.at[pl.ds(r0, rpt)])

        @pl.when(cid == 1)
        def _():
            pltpu.sync_copy(acc.at[pl.ds(r0, rpt)], outB.at[pl.ds(r0, rpt)])

    return spmm


def kernel(x, edge_index, gcn_weight, sct_weight, res_weight,
           W_hyb, b_hyb, W_res, b_res):
    n = x.shape[0]
    e = edge_index.shape[1]

    nch = -(-e // (N_TILES * CHUNK))
    nch = -(-nch // 8) * 8
    pad = nch * N_TILES * CHUNK - e
    zi = jnp.zeros((pad,), jnp.int32)
    zf = jnp.zeros((pad,), jnp.float32)
    src = jnp.concatenate([edge_index[0], zi]).reshape(-1, CHUNK)
    dst = jnp.concatenate([edge_index[1], zi]).reshape(-1, CHUNK)
    wg = jnp.concatenate([gcn_weight, zf]).reshape(-1, CHUNK)
    ws = jnp.concatenate([sct_weight, zf]).reshape(-1, CHUNK)
    wr = jnp.concatenate([res_weight, zf]).reshape(-1, CHUNK)

    d_in = x.shape[1]
    z8 = jnp.zeros((d_in, 8), jnp.float32)
    W_all = jnp.concatenate(
        [W_hyb[3], W_hyb[4], W_hyb[5], z8,
         W_hyb[0], W_hyb[1], W_hyb[2], z8], axis=1)
    zb8 = jnp.zeros((8,), jnp.float32)
    b_all = jnp.concatenate(
        [b_hyb[3], b_hyb[4], b_hyb[5], zb8,
         b_hyb[0], b_hyb[1], b_hyb[2], zb8]).reshape(1, 64)

    hg, hs = _dense_in(x, W_all, b_all)

    npad = -(-n // (N_TILES * 8)) * (N_TILES * 8)

    g1, s1 = _make_spmm(32, 32, 0, npad, nch)(hg, hs, wg, ws, src, dst)
    g2, s2 = _make_spmm(16, 16, 0, npad, nch)(
        g1[:, 8:24], s1[:, 8:24], wg, ws, src, dst)
    g3, s3 = _make_spmm(16, 16, 0, npad, nch)(g2, s2, wg, ws, src, dst)

    hA, hB = _dense_mid(g1, g2, g3, s1, s2, s3)

    pA, pB = _make_spmm(32, 32, 0, npad, nch)(hA, hB, wr, wr, src, dst)
    return _dense_out(n, pA, pB, W_res, b_res.reshape(1, -1))
